# Initial kernel scaffold; baseline (speedup 1.0000x reference)
import jax
import jax.numpy as jnp
from jax.experimental import pallas as pl


def kernel(preds, targets):
    raise NotImplementedError("write your pallas kernel here")


# speedup vs baseline: 2.4760x; 2.4760x over previous
import math

import jax
import jax.numpy as jnp
from jax.experimental import pallas as pl
from jax.experimental.pallas import tpu as pltpu

_NC = 80
_IOU_THR = 0.45
_HYP_CLS = 0.5
_HYP_BOX = 7.5
_EPS = 1e-7

_N = 5000
_M = 2000
_R = 256
_NPAD = 5120
_MPAD = 2048


def _atan(x):
    t = jnp.abs(x)
    c1 = t > 2.414213562373095
    c2 = t > 0.4142135623730951
    xr = jnp.where(c1, -1.0 / jnp.maximum(t, 1e-30),
                   jnp.where(c2, (t - 1.0) / (t + 1.0), t))
    y0 = jnp.where(c1, math.pi / 2, jnp.where(c2, math.pi / 4, 0.0))
    z = xr * xr
    p = (((8.05374449538e-2 * z - 1.38776856032e-1) * z
          + 1.99777106478e-1) * z - 3.33329491539e-1) * z * xr + xr
    return jnp.sign(x) * (y0 + p)


def _body(preds_ref, tgt_ref, out_ref, macc_ref, sacc_ref):
    i = pl.program_id(0)
    nt = pl.num_programs(0)

    @pl.when(i == 0)
    def _init():
        macc_ref[0:1, :] = jnp.zeros((1, _MPAD), jnp.float32)
        for k in range(4):
            sacc_ref[k] = 0.0

    P = preds_ref[...]
    px1 = P[:, 0:1]
    py1 = P[:, 1:2]
    px2 = P[:, 2:3]
    py2 = P[:, 3:4]
    s = P[:, 4:5]
    pcls = P[:, 5:6]

    T = tgt_ref[...]
    tx1 = T[0:1, :]
    ty1 = T[1:2, :]
    tx2 = T[2:3, :]
    ty2 = T[3:4, :]
    tcls = T[4:5, :]

    col = jax.lax.broadcasted_iota(jnp.int32, (1, _MPAD), 1).astype(jnp.float32)
    colmask = col < float(_M)

    a1 = (px2 - px1) * (py2 - py1)
    a2 = (tx2 - tx1) * (ty2 - ty1)
    iw = jnp.maximum(jnp.minimum(px2, tx2) - jnp.maximum(px1, tx1), 0.0)
    ih = jnp.maximum(jnp.minimum(py2, ty2) - jnp.maximum(py1, ty1), 0.0)
    inter = iw * ih
    iou = inter / (a1 + a2 - inter + _EPS)
    iou = jnp.where(colmask, iou, -1.0)

    mx = jnp.max(iou, axis=1, keepdims=True)
    idxv = jnp.min(jnp.where(iou == mx, col, float(2 * _MPAD)),
                   axis=1, keepdims=True)
    ohf = (col == idxv).astype(jnp.float32)

    grow = (jax.lax.broadcasted_iota(jnp.int32, (_R, 1), 0).astype(jnp.float32)
            + jnp.float32(_R) * i.astype(jnp.float32))
    valid = grow < float(_N)
    keep = (mx > _IOU_THR) & valid
    kf = keep.astype(jnp.float32)

    m_x1 = jnp.sum(ohf * tx1, axis=1, keepdims=True)
    m_y1 = jnp.sum(ohf * ty1, axis=1, keepdims=True)
    m_x2 = jnp.sum(ohf * tx2, axis=1, keepdims=True)
    m_y2 = jnp.sum(ohf * ty2, axis=1, keepdims=True)
    m_cls = jnp.sum(ohf * tcls, axis=1, keepdims=True)

    s_pos = s > 0.0
    logs = jnp.where(s_pos, jnp.log(jnp.where(s_pos, s, 1.0)), -100.0)
    log1ms = jnp.maximum(jnp.log1p(-s), -100.0)
    same = m_cls == pcls
    bce_row = jnp.where(keep, jnp.where(same, -logs, 100.0 - log1ms), 0.0)
    unm_row = jnp.where((~keep) & valid, logs, 0.0)

    w1 = px2 - px1
    h1 = py2 - py1 + _EPS
    w2 = m_x2 - m_x1
    h2 = m_y2 - m_y1 + _EPS
    inter_c = (jnp.maximum(jnp.minimum(px2, m_x2) - jnp.maximum(px1, m_x1), 0.0)
               * jnp.maximum(jnp.minimum(py2, m_y2) - jnp.maximum(py1, m_y1), 0.0))
    union_c = w1 * h1 + w2 * h2 - inter_c + _EPS
    iou_c = inter_c / union_c
    cw = jnp.maximum(px2, m_x2) - jnp.minimum(px1, m_x1)
    ch = jnp.maximum(py2, m_y2) - jnp.minimum(py1, m_y1)
    c2 = cw * cw + ch * ch + _EPS
    rho2 = ((m_x1 + m_x2 - px1 - px2) ** 2 + (m_y1 + m_y2 - py1 - py2) ** 2) / 4.0
    datan = _atan((w2 * h1 - w1 * h2) / (h1 * h2 + w1 * w2))
    v = (4.0 / math.pi ** 2) * datan ** 2
    alpha = v / (v - iou_c + (1.0 + _EPS))
    ciou = iou_c - (rho2 / c2 + v * alpha)
    bbox_row = jnp.where(keep, 1.0 - ciou, 0.0)

    sacc_ref[0] += jnp.sum(kf)
    sacc_ref[1] += jnp.sum(bce_row)
    sacc_ref[2] += jnp.sum(unm_row)
    sacc_ref[3] += jnp.sum(bbox_row)
    colm = jnp.max(ohf * kf, axis=0, keepdims=True)
    macc_ref[0:1, :] = jnp.maximum(macc_ref[0:1, :], colm)

    @pl.when(i == nt - 1)
    def _fin():
        nk = jnp.maximum(sacc_ref[0], 1.0)
        matched_cnt = jnp.sum(macc_ref[0:1, :])
        cls_loss = (sacc_ref[1] / (nk * float(_NC)) - sacc_ref[2]
                    + (float(_M) - matched_cnt))
        bbox_loss = sacc_ref[3] / nk
        total = _HYP_CLS * cls_loss + _HYP_BOX * bbox_loss
        out_ref[...] = jnp.full((8, 128), total, jnp.float32)


def kernel(preds, targets):
    preds_pad = jnp.zeros((_NPAD, 8), jnp.float32)
    preds_pad = preds_pad.at[:_N, :6].set(preds.astype(jnp.float32))
    tgt_t = jnp.zeros((8, _MPAD), jnp.float32)
    tgt_t = tgt_t.at[:5, :_M].set(targets.astype(jnp.float32).T)

    nt = _NPAD // _R
    out = pl.pallas_call(
        _body,
        grid=(nt,),
        in_specs=[
            pl.BlockSpec((_R, 8), lambda i: (i, 0)),
            pl.BlockSpec((8, _MPAD), lambda i: (0, 0)),
        ],
        out_specs=pl.BlockSpec((8, 128), lambda i: (0, 0)),
        out_shape=jax.ShapeDtypeStruct((8, 128), jnp.float32),
        scratch_shapes=[
            pltpu.VMEM((8, _MPAD), jnp.float32),
            pltpu.SMEM((8,), jnp.float32),
        ],
        compiler_params=pltpu.CompilerParams(
            dimension_semantics=("arbitrary",),
        ),
    )(preds_pad, tgt_t)
    return out[0, 0]


# speedup vs baseline: 2.8384x; 1.1464x over previous
import math

import jax
import jax.numpy as jnp
from jax.experimental import pallas as pl
from jax.experimental.pallas import tpu as pltpu

_NC = 80
_IOU_THR = 0.45
_HYP_CLS = 0.5
_HYP_BOX = 7.5
_EPS = 1e-7

_N = 5000
_M = 2000
_R = 256
_NPAD = 5120
_MPAD = 2048


def _atan(x):
    t = jnp.abs(x)
    c1 = t > 2.414213562373095
    c2 = t > 0.4142135623730951
    xr = jnp.where(c1, -1.0 / jnp.maximum(t, 1e-30),
                   jnp.where(c2, (t - 1.0) / (t + 1.0), t))
    y0 = jnp.where(c1, math.pi / 2, jnp.where(c2, math.pi / 4, 0.0))
    z = xr * xr
    p = (((8.05374449538e-2 * z - 1.38776856032e-1) * z
          + 1.99777106478e-1) * z - 3.33329491539e-1) * z * xr + xr
    return jnp.sign(x) * (y0 + p)


def _body(preds_ref, tgt_ref, tgt2_ref, out_ref, macc_ref, sacc_ref):
    i = pl.program_id(0)
    nt = pl.num_programs(0)

    @pl.when(i == 0)
    def _init():
        macc_ref[0:1, :] = jnp.zeros((1, _MPAD), jnp.float32)
        for k in range(4):
            sacc_ref[k] = 0.0

    P = preds_ref[...]
    px1 = P[:, 0:1]
    py1 = P[:, 1:2]
    px2 = P[:, 2:3]
    py2 = P[:, 3:4]
    s = P[:, 4:5]
    pcls = P[:, 5:6]

    T = tgt_ref[...]
    tx1 = T[0:1, :]
    ty1 = T[1:2, :]
    tx2 = T[2:3, :]
    ty2 = T[3:4, :]
    tcls = T[4:5, :]

    col = jax.lax.broadcasted_iota(jnp.int32, (1, _MPAD), 1).astype(jnp.float32)
    colmask = col < float(_M)

    a1 = (px2 - px1) * (py2 - py1)
    a2 = (tx2 - tx1) * (ty2 - ty1)
    iw = jnp.maximum(jnp.minimum(px2, tx2) - jnp.maximum(px1, tx1), 0.0)
    ih = jnp.maximum(jnp.minimum(py2, ty2) - jnp.maximum(py1, ty1), 0.0)
    inter = iw * ih
    iou = inter / (a1 + a2 - inter + _EPS)
    iou = jnp.where(colmask, iou, -1.0)

    mx = jnp.max(iou, axis=1, keepdims=True)
    idxv = jnp.min(jnp.where(iou == mx, col, float(2 * _MPAD)),
                   axis=1, keepdims=True)
    ohf = (col == idxv).astype(jnp.float32)

    grow = (jax.lax.broadcasted_iota(jnp.int32, (_R, 1), 0).astype(jnp.float32)
            + jnp.float32(_R) * i.astype(jnp.float32))
    valid = grow < float(_N)
    keep = (mx > _IOU_THR) & valid
    kf = keep.astype(jnp.float32)

    m = jax.lax.dot_general(ohf, tgt2_ref[...], (((1,), (0,)), ((), ())),
                            preferred_element_type=jnp.float32)
    m_x1 = m[:, 0:1]
    m_y1 = m[:, 1:2]
    m_x2 = m[:, 2:3]
    m_y2 = m[:, 3:4]
    m_cls = m[:, 4:5]

    s_pos = s > 0.0
    logs = jnp.where(s_pos, jnp.log(jnp.where(s_pos, s, 1.0)), -100.0)
    log1ms = jnp.maximum(jnp.log1p(-s), -100.0)
    same = m_cls == pcls
    bce_row = jnp.where(keep, jnp.where(same, -logs, 100.0 - log1ms), 0.0)
    unm_row = jnp.where((~keep) & valid, logs, 0.0)

    w1 = px2 - px1
    h1 = py2 - py1 + _EPS
    w2 = m_x2 - m_x1
    h2 = m_y2 - m_y1 + _EPS
    inter_c = (jnp.maximum(jnp.minimum(px2, m_x2) - jnp.maximum(px1, m_x1), 0.0)
               * jnp.maximum(jnp.minimum(py2, m_y2) - jnp.maximum(py1, m_y1), 0.0))
    union_c = w1 * h1 + w2 * h2 - inter_c + _EPS
    iou_c = inter_c / union_c
    cw = jnp.maximum(px2, m_x2) - jnp.minimum(px1, m_x1)
    ch = jnp.maximum(py2, m_y2) - jnp.minimum(py1, m_y1)
    c2 = cw * cw + ch * ch + _EPS
    rho2 = ((m_x1 + m_x2 - px1 - px2) ** 2 + (m_y1 + m_y2 - py1 - py2) ** 2) / 4.0
    datan = _atan((w2 * h1 - w1 * h2) / (h1 * h2 + w1 * w2))
    v = (4.0 / math.pi ** 2) * datan ** 2
    alpha = v / (v - iou_c + (1.0 + _EPS))
    ciou = iou_c - (rho2 / c2 + v * alpha)
    bbox_row = jnp.where(keep, 1.0 - ciou, 0.0)

    sacc_ref[0] += jnp.sum(kf)
    sacc_ref[1] += jnp.sum(bce_row)
    sacc_ref[2] += jnp.sum(unm_row)
    sacc_ref[3] += jnp.sum(bbox_row)
    colm = jax.lax.dot_general(kf, ohf, (((0,), (0,)), ((), ())),
                               preferred_element_type=jnp.float32)
    macc_ref[0:1, :] += colm

    @pl.when(i == nt - 1)
    def _fin():
        nk = jnp.maximum(sacc_ref[0], 1.0)
        matched_cnt = jnp.sum((macc_ref[0:1, :] > 0.0).astype(jnp.float32))
        cls_loss = (sacc_ref[1] / (nk * float(_NC)) - sacc_ref[2]
                    + (float(_M) - matched_cnt))
        bbox_loss = sacc_ref[3] / nk
        total = _HYP_CLS * cls_loss + _HYP_BOX * bbox_loss
        out_ref[...] = jnp.full((8, 128), total, jnp.float32)


def kernel(preds, targets):
    preds_pad = jnp.zeros((_NPAD, 8), jnp.float32)
    preds_pad = preds_pad.at[:_N, :6].set(preds.astype(jnp.float32))
    tgt_t = jnp.zeros((8, _MPAD), jnp.float32)
    tgt_t = tgt_t.at[:5, :_M].set(targets.astype(jnp.float32).T)
    tgt2 = jnp.zeros((_MPAD, 8), jnp.float32)
    tgt2 = tgt2.at[:_M, :5].set(targets.astype(jnp.float32))

    nt = _NPAD // _R
    out = pl.pallas_call(
        _body,
        grid=(nt,),
        in_specs=[
            pl.BlockSpec((_R, 8), lambda i: (i, 0)),
            pl.BlockSpec((8, _MPAD), lambda i: (0, 0)),
            pl.BlockSpec((_MPAD, 8), lambda i: (0, 0)),
        ],
        out_specs=pl.BlockSpec((8, 128), lambda i: (0, 0)),
        out_shape=jax.ShapeDtypeStruct((8, 128), jnp.float32),
        scratch_shapes=[
            pltpu.VMEM((8, _MPAD), jnp.float32),
            pltpu.SMEM((8,), jnp.float32),
        ],
        compiler_params=pltpu.CompilerParams(
            dimension_semantics=("arbitrary",),
        ),
    )(preds_pad, tgt_t, tgt2)
    return out[0, 0]


# speedup vs baseline: 3.0476x; 1.0737x over previous
import math

import jax
import jax.numpy as jnp
from jax.experimental import pallas as pl
from jax.experimental.pallas import tpu as pltpu

_NC = 80
_IOU_THR = 0.45
_HYP_CLS = 0.5
_HYP_BOX = 7.5
_EPS = 1e-7

_N = 5000
_M = 2000
_R = 512
_NPAD = 5120
_MPAD = 2048


def _atan(x):
    t = jnp.abs(x)
    c1 = t > 2.414213562373095
    c2 = t > 0.4142135623730951
    xr = jnp.where(c1, -1.0 / jnp.maximum(t, 1e-30),
                   jnp.where(c2, (t - 1.0) / (t + 1.0), t))
    y0 = jnp.where(c1, math.pi / 2, jnp.where(c2, math.pi / 4, 0.0))
    z = xr * xr
    p = (((8.05374449538e-2 * z - 1.38776856032e-1) * z
          + 1.99777106478e-1) * z - 3.33329491539e-1) * z * xr + xr
    return jnp.sign(x) * (y0 + p)


def _body(preds_ref, tgt_ref, tgt2_ref, out_ref, macc_ref, sacc_ref):
    i = pl.program_id(0)
    nt = pl.num_programs(0)

    @pl.when(i == 0)
    def _init():
        macc_ref[0:1, :] = jnp.zeros((1, _MPAD), jnp.float32)
        for k in range(4):
            sacc_ref[k] = 0.0

    P = preds_ref[...]
    px1 = P[:, 0:1]
    py1 = P[:, 1:2]
    px2 = P[:, 2:3]
    py2 = P[:, 3:4]
    s = P[:, 4:5]
    pcls = P[:, 5:6]

    T = tgt_ref[...]
    tx1 = T[0:1, :]
    ty1 = T[1:2, :]
    tx2 = T[2:3, :]
    ty2 = T[3:4, :]
    tcls = T[4:5, :]

    col = jax.lax.broadcasted_iota(jnp.int32, (1, _MPAD), 1).astype(jnp.float32)

    a1 = (px2 - px1) * (py2 - py1)
    a2 = (tx2 - tx1) * (ty2 - ty1)
    iw = jnp.maximum(jnp.minimum(px2, tx2) - jnp.maximum(px1, tx1), 0.0)
    ih = jnp.maximum(jnp.minimum(py2, ty2) - jnp.maximum(py1, ty1), 0.0)
    inter = iw * ih
    iou = inter / (a1 + a2 - inter + _EPS)

    mx = jnp.max(iou, axis=1, keepdims=True)
    idxv = jnp.min(jnp.where(iou == mx, col, float(2 * _MPAD)),
                   axis=1, keepdims=True)
    ohf = (col == idxv).astype(jnp.float32)

    grow = (jax.lax.broadcasted_iota(jnp.int32, (_R, 1), 0).astype(jnp.float32)
            + jnp.float32(_R) * i.astype(jnp.float32))
    valid = grow < float(_N)
    keep = (mx > _IOU_THR) & valid
    kf = keep.astype(jnp.float32)

    m = jax.lax.dot_general(ohf, tgt2_ref[...], (((1,), (0,)), ((), ())),
                            preferred_element_type=jnp.float32)
    m_x1 = m[:, 0:1]
    m_y1 = m[:, 1:2]
    m_x2 = m[:, 2:3]
    m_y2 = m[:, 3:4]
    m_cls = m[:, 4:5]

    s_pos = s > 0.0
    logs = jnp.where(s_pos, jnp.log(jnp.where(s_pos, s, 1.0)), -100.0)
    log1ms = jnp.maximum(jnp.log1p(-s), -100.0)
    same = m_cls == pcls
    bce_row = jnp.where(keep, jnp.where(same, -logs, 100.0 - log1ms), 0.0)
    unm_row = jnp.where((~keep) & valid, logs, 0.0)

    w1 = px2 - px1
    h1 = py2 - py1 + _EPS
    w2 = m_x2 - m_x1
    h2 = m_y2 - m_y1 + _EPS
    inter_c = (jnp.maximum(jnp.minimum(px2, m_x2) - jnp.maximum(px1, m_x1), 0.0)
               * jnp.maximum(jnp.minimum(py2, m_y2) - jnp.maximum(py1, m_y1), 0.0))
    union_c = w1 * h1 + w2 * h2 - inter_c + _EPS
    iou_c = inter_c / union_c
    cw = jnp.maximum(px2, m_x2) - jnp.minimum(px1, m_x1)
    ch = jnp.maximum(py2, m_y2) - jnp.minimum(py1, m_y1)
    c2 = cw * cw + ch * ch + _EPS
    rho2 = ((m_x1 + m_x2 - px1 - px2) ** 2 + (m_y1 + m_y2 - py1 - py2) ** 2) / 4.0
    datan = _atan((w2 * h1 - w1 * h2) / (h1 * h2 + w1 * w2))
    v = (4.0 / math.pi ** 2) * datan ** 2
    alpha = v / (v - iou_c + (1.0 + _EPS))
    ciou = iou_c - (rho2 / c2 + v * alpha)
    bbox_row = jnp.where(keep, 1.0 - ciou, 0.0)

    sacc_ref[0] += jnp.sum(kf)
    sacc_ref[1] += jnp.sum(bce_row)
    sacc_ref[2] += jnp.sum(unm_row)
    sacc_ref[3] += jnp.sum(bbox_row)
    colm = jax.lax.dot_general(kf, ohf, (((0,), (0,)), ((), ())),
                               preferred_element_type=jnp.float32)
    macc_ref[0:1, :] += colm

    @pl.when(i == nt - 1)
    def _fin():
        nk = jnp.maximum(sacc_ref[0], 1.0)
        matched_cnt = jnp.sum((macc_ref[0:1, :] > 0.0).astype(jnp.float32))
        cls_loss = (sacc_ref[1] / (nk * float(_NC)) - sacc_ref[2]
                    + (float(_M) - matched_cnt))
        bbox_loss = sacc_ref[3] / nk
        total = _HYP_CLS * cls_loss + _HYP_BOX * bbox_loss
        out_ref[...] = jnp.full((8, 128), total, jnp.float32)


def kernel(preds, targets):
    preds_pad = jnp.zeros((_NPAD, 8), jnp.float32)
    preds_pad = preds_pad.at[:_N, :6].set(preds.astype(jnp.float32))
    tgt_t = jnp.zeros((8, _MPAD), jnp.float32)
    tgt_t = tgt_t.at[:5, :_M].set(targets.astype(jnp.float32).T)
    tgt2 = jnp.zeros((_MPAD, 8), jnp.float32)
    tgt2 = tgt2.at[:_M, :5].set(targets.astype(jnp.float32))

    nt = _NPAD // _R
    out = pl.pallas_call(
        _body,
        grid=(nt,),
        in_specs=[
            pl.BlockSpec((_R, 8), lambda i: (i, 0)),
            pl.BlockSpec((8, _MPAD), lambda i: (0, 0)),
            pl.BlockSpec((_MPAD, 8), lambda i: (0, 0)),
        ],
        out_specs=pl.BlockSpec((8, 128), lambda i: (0, 0)),
        out_shape=jax.ShapeDtypeStruct((8, 128), jnp.float32),
        scratch_shapes=[
            pltpu.VMEM((8, _MPAD), jnp.float32),
            pltpu.SMEM((8,), jnp.float32),
        ],
        compiler_params=pltpu.CompilerParams(
            dimension_semantics=("arbitrary",),
        ),
    )(preds_pad, tgt_t, tgt2)
    return out[0, 0]


# speedup vs baseline: 3.3193x; 1.0892x over previous
import math

import jax
import jax.numpy as jnp
from jax.experimental import pallas as pl
from jax.experimental.pallas import tpu as pltpu

_NC = 80
_IOU_THR = 0.45
_HYP_CLS = 0.5
_HYP_BOX = 7.5
_EPS = 1e-7

_N = 5000
_M = 2000
_R = 1000
_NPAD = 5000
_MPAD = 2048


def _atan(x):
    t = jnp.abs(x)
    c1 = t > 2.414213562373095
    c2 = t > 0.4142135623730951
    xr = jnp.where(c1, -1.0 / jnp.maximum(t, 1e-30),
                   jnp.where(c2, (t - 1.0) / (t + 1.0), t))
    y0 = jnp.where(c1, math.pi / 2, jnp.where(c2, math.pi / 4, 0.0))
    z = xr * xr
    p = (((8.05374449538e-2 * z - 1.38776856032e-1) * z
          + 1.99777106478e-1) * z - 3.33329491539e-1) * z * xr + xr
    return jnp.sign(x) * (y0 + p)


def _body(preds_ref, tgt_ref, tgt2_ref, out_ref, macc_ref, sacc_ref):
    i = pl.program_id(0)
    nt = pl.num_programs(0)

    @pl.when(i == 0)
    def _init():
        macc_ref[0:1, :] = jnp.zeros((1, _MPAD), jnp.float32)
        for k in range(4):
            sacc_ref[k] = 0.0

    P = preds_ref[...]
    px1 = P[:, 0:1]
    py1 = P[:, 1:2]
    px2 = P[:, 2:3]
    py2 = P[:, 3:4]
    s = P[:, 4:5]
    pcls = P[:, 5:6]

    T = tgt_ref[...]
    tx1 = T[0:1, :]
    ty1 = T[1:2, :]
    tx2 = T[2:3, :]
    ty2 = T[3:4, :]
    a2e = T[5:6, :]

    col = jax.lax.broadcasted_iota(jnp.int32, (1, _MPAD), 1).astype(jnp.float32)

    a1 = (px2 - px1) * (py2 - py1)
    iw = jnp.maximum(jnp.minimum(px2, tx2) - jnp.maximum(px1, tx1), 0.0)
    ih = jnp.maximum(jnp.minimum(py2, ty2) - jnp.maximum(py1, ty1), 0.0)
    inter = iw * ih
    iou = inter / ((a1 + a2e) - inter)

    mx = jnp.max(iou, axis=1, keepdims=True)
    idxv = jnp.min(jnp.where(iou == mx, col, float(2 * _MPAD)),
                   axis=1, keepdims=True)
    ohf = (col == idxv).astype(jnp.float32)

    keep = mx > _IOU_THR
    kf = keep.astype(jnp.float32)

    m = jax.lax.dot_general(ohf, tgt2_ref[...], (((1,), (0,)), ((), ())),
                            preferred_element_type=jnp.float32)
    m_x1 = m[:, 0:1]
    m_y1 = m[:, 1:2]
    m_x2 = m[:, 2:3]
    m_y2 = m[:, 3:4]
    m_cls = m[:, 4:5]

    s_pos = s > 0.0
    logs = jnp.where(s_pos, jnp.log(jnp.where(s_pos, s, 1.0)), -100.0)
    log1ms = jnp.maximum(jnp.log1p(-s), -100.0)
    same = m_cls == pcls
    bce_row = jnp.where(keep, jnp.where(same, -logs, 100.0 - log1ms), 0.0)
    unm_row = jnp.where(~keep, logs, 0.0)

    w1 = px2 - px1
    h1 = py2 - py1 + _EPS
    w2 = m_x2 - m_x1
    h2 = m_y2 - m_y1 + _EPS
    inter_c = (jnp.maximum(jnp.minimum(px2, m_x2) - jnp.maximum(px1, m_x1), 0.0)
               * jnp.maximum(jnp.minimum(py2, m_y2) - jnp.maximum(py1, m_y1), 0.0))
    union_c = w1 * h1 + w2 * h2 - inter_c + _EPS
    iou_c = inter_c / union_c
    cw = jnp.maximum(px2, m_x2) - jnp.minimum(px1, m_x1)
    ch = jnp.maximum(py2, m_y2) - jnp.minimum(py1, m_y1)
    c2 = cw * cw + ch * ch + _EPS
    rho2 = ((m_x1 + m_x2 - px1 - px2) ** 2 + (m_y1 + m_y2 - py1 - py2) ** 2) / 4.0
    datan = _atan((w2 * h1 - w1 * h2) / (h1 * h2 + w1 * w2))
    v = (4.0 / math.pi ** 2) * datan ** 2
    alpha = v / (v - iou_c + (1.0 + _EPS))
    ciou = iou_c - (rho2 / c2 + v * alpha)
    bbox_row = jnp.where(keep, 1.0 - ciou, 0.0)

    sacc_ref[0] += jnp.sum(kf)
    sacc_ref[1] += jnp.sum(bce_row)
    sacc_ref[2] += jnp.sum(unm_row)
    sacc_ref[3] += jnp.sum(bbox_row)
    colm = jax.lax.dot_general(kf, ohf, (((0,), (0,)), ((), ())),
                               preferred_element_type=jnp.float32)
    macc_ref[0:1, :] += colm

    @pl.when(i == nt - 1)
    def _fin():
        nk = jnp.maximum(sacc_ref[0], 1.0)
        matched_cnt = jnp.sum((macc_ref[0:1, :] > 0.0).astype(jnp.float32))
        cls_loss = (sacc_ref[1] / (nk * float(_NC)) - sacc_ref[2]
                    + (float(_M) - matched_cnt))
        bbox_loss = sacc_ref[3] / nk
        total = _HYP_CLS * cls_loss + _HYP_BOX * bbox_loss
        out_ref[...] = jnp.full((8, 128), total, jnp.float32)


def kernel(preds, targets):
    preds_pad = jnp.zeros((_NPAD, 8), jnp.float32)
    preds_pad = preds_pad.at[:_N, :6].set(preds.astype(jnp.float32))
    tf = targets.astype(jnp.float32)
    tgt_t = jnp.zeros((8, _MPAD), jnp.float32)
    tgt_t = tgt_t.at[:5, :_M].set(tf.T)
    a2 = (tf[:, 2] - tf[:, 0]) * (tf[:, 3] - tf[:, 1])
    tgt_t = tgt_t.at[5, :].set(_EPS)
    tgt_t = tgt_t.at[5, :_M].add(a2)
    tgt2 = jnp.zeros((_MPAD, 8), jnp.float32)
    tgt2 = tgt2.at[:_M, :5].set(targets.astype(jnp.float32))

    nt = _NPAD // _R
    out = pl.pallas_call(
        _body,
        grid=(nt,),
        in_specs=[
            pl.BlockSpec((_R, 8), lambda i: (i, 0)),
            pl.BlockSpec((8, _MPAD), lambda i: (0, 0)),
            pl.BlockSpec((_MPAD, 8), lambda i: (0, 0)),
        ],
        out_specs=pl.BlockSpec((8, 128), lambda i: (0, 0)),
        out_shape=jax.ShapeDtypeStruct((8, 128), jnp.float32),
        scratch_shapes=[
            pltpu.VMEM((8, _MPAD), jnp.float32),
            pltpu.SMEM((8,), jnp.float32),
        ],
        compiler_params=pltpu.CompilerParams(
            dimension_semantics=("arbitrary",),
        ),
    )(preds_pad, tgt_t, tgt2)
    return out[0, 0]
